# baseline (speedup 1.0000x reference)
import functools

import jax
import jax.numpy as jnp
from jax import lax
from jax.experimental import pallas as pl
from jax.experimental.pallas import tpu as pltpu
from jax.experimental.pallas import tpu_sc as plsc

NUM_PIXELS = 1000 * 1000
C = 64
EMBED_DIM = 128
N_IDX = 16384

_info = plsc.get_sparse_core_info()
NC, NS = _info.num_cores, _info.num_subcores
NW = NC * NS
BPW = N_IDX // NW
CHUNK = 128
NCHUNK = BPW // CHUNK


def _sc_gather(table, o_idx3, d_idx3):
    mesh = plsc.VectorSubcoreMesh(core_axis_name="c", subcore_axis_name="s")

    @functools.partial(
        pl.kernel,
        mesh=mesh,
        compiler_params=pltpu.CompilerParams(use_tc_tiling_on_sc=False),
        out_type=[
            jax.ShapeDtypeStruct((N_IDX, C), jnp.float32),
            jax.ShapeDtypeStruct((N_IDX, C), jnp.float32),
        ],
        scratch_types=[
            pltpu.VMEM((NCHUNK, CHUNK), jnp.int32),
            pltpu.VMEM((NCHUNK, CHUNK), jnp.int32),
            pltpu.VMEM((BPW, C), jnp.float32),
            pltpu.VMEM((BPW, C), jnp.float32),
            pltpu.SemaphoreType.DMA,
            pltpu.SemaphoreType.DMA,
        ],
    )
    def k(table_h, o_idx_h, d_idx_h, o_out_h, d_out_h,
          o_idx_v, d_idx_v, o_rows_v, d_rows_v, o_sem, d_sem):
        wid = lax.axis_index("s") * NC + lax.axis_index("c")
        base = wid * BPW
        pltpu.sync_copy(o_idx_h.at[wid], o_idx_v)
        pltpu.sync_copy(d_idx_h.at[wid], d_idx_v)
        o_copies = [
            pltpu.async_copy(table_h.at[o_idx_v.at[j]],
                             o_rows_v.at[pl.ds(j * CHUNK, CHUNK)], o_sem)
            for j in range(NCHUNK)
        ]
        d_copies = [
            pltpu.async_copy(table_h.at[d_idx_v.at[j]],
                             d_rows_v.at[pl.ds(j * CHUNK, CHUNK)], d_sem)
            for j in range(NCHUNK)
        ]
        for cp in o_copies:
            cp.wait()
        pltpu.sync_copy(o_rows_v, o_out_h.at[pl.ds(base, BPW)])
        for cp in d_copies:
            cp.wait()
        pltpu.sync_copy(d_rows_v, d_out_h.at[pl.ds(base, BPW)])

    return k(table, o_idx3, d_idx3)


def _tc_mlp(o_rows, d_rows, W, b2):
    Br = 1024
    grid = (N_IDX // Br,)

    def body(o_ref, d_ref, w_ref, b_ref, oo_ref, do_ref):
        w = w_ref[...]
        bb = b_ref[...]
        oo_ref[...] = jnp.maximum(
            jnp.dot(o_ref[...], w, preferred_element_type=jnp.float32) + bb, 0.0)
        do_ref[...] = jnp.maximum(
            jnp.dot(d_ref[...], w, preferred_element_type=jnp.float32) + bb, 0.0)

    return pl.pallas_call(
        body,
        grid=grid,
        in_specs=[
            pl.BlockSpec((Br, C), lambda i: (i, 0)),
            pl.BlockSpec((Br, C), lambda i: (i, 0)),
            pl.BlockSpec((C, EMBED_DIM), lambda i: (0, 0)),
            pl.BlockSpec((1, EMBED_DIM), lambda i: (0, 0)),
        ],
        out_specs=[
            pl.BlockSpec((Br, EMBED_DIM), lambda i: (i, 0)),
            pl.BlockSpec((Br, EMBED_DIM), lambda i: (i, 0)),
        ],
        out_shape=[
            jax.ShapeDtypeStruct((N_IDX, EMBED_DIM), jnp.float32),
            jax.ShapeDtypeStruct((N_IDX, EMBED_DIM), jnp.float32),
        ],
    )(o_rows, d_rows, W, b2)


@jax.jit
def _impl(grid_features, W, b, o_indices, d_indices):
    o_idx3 = o_indices.reshape(NW, NCHUNK, CHUNK)
    d_idx3 = d_indices.reshape(NW, NCHUNK, CHUNK)
    o_rows, d_rows = _sc_gather(grid_features, o_idx3, d_idx3)
    return _tc_mlp(o_rows, d_rows, W, b.reshape(1, EMBED_DIM))


def kernel(grid_features, W, b, o_indices, d_indices):
    return _impl(grid_features, W, b, o_indices, d_indices)
